# baseline (speedup 1.0000x reference)
import jax
import jax.numpy as jnp
from jax.experimental import pallas as pl
from jax.experimental.pallas import tpu as pltpu
from jax.experimental.pallas import tpu_sc as plsc

_FEATURES = 64
_ROWS_PER_STEP = 4


def kernel(inputs, embedding):
    batch, seq = inputs.shape
    num_indices = batch * seq
    pad_cols = 128 - seq
    filler = (
        jax.lax.broadcasted_iota(jnp.int32, (batch, pad_cols), 0) * pad_cols
        + jax.lax.broadcasted_iota(jnp.int32, (batch, pad_cols), 1)
    )
    idx = jnp.concatenate([inputs.astype(jnp.int32), filler], axis=1)
    idx = idx.reshape(batch * 128)
    mesh = plsc.VectorSubcoreMesh(
        core_axis_name="core", subcore_axis_name="subcore"
    )

    seq_pad = 56

    @pl.kernel(
        out_type=jax.ShapeDtypeStruct(
            (batch * seq_pad, _FEATURES), embedding.dtype
        ),
        mesh=mesh,
        scratch_types=[pltpu.SemaphoreType.DMA],
        compiler_params=pltpu.CompilerParams(use_tc_tiling_on_sc=False),
    )
    def _gather(x_hbm, i_hbm, o_hbm, sem):
        def body(i_vmem, o_vmem):
            copies = [
                pltpu.async_copy(
                    x_hbm.at[i_vmem.at[pl.ds(r * 128, seq_pad)]],
                    o_vmem.at[pl.ds(r * seq_pad, seq_pad)],
                    sem,
                )
                for r in range(_ROWS_PER_STEP)
            ]
            for c in copies:
                c.wait()

        pltpu.emit_pipeline(
            body,
            grid=(batch // _ROWS_PER_STEP,),
            in_specs=[
                pl.BlockSpec(
                    (_ROWS_PER_STEP * 128,), index_map=lambda i: (i,)
                )
            ],
            out_specs=[
                pl.BlockSpec(
                    (_ROWS_PER_STEP * seq_pad, _FEATURES),
                    index_map=lambda i: (i, 0),
                )
            ],
            core_axis_name=("core", "subcore"),
            dimension_semantics=(pltpu.PARALLEL,),
        )(i_hbm, o_hbm)

    out = _gather(embedding, idx)
    return out.reshape(batch, seq_pad, _FEATURES)[:, :seq, :]


# speedup vs baseline: 1.1707x; 1.1707x over previous
import jax
import jax.numpy as jnp
from jax.experimental import pallas as pl
from jax.experimental.pallas import tpu as pltpu
from jax.experimental.pallas import tpu_sc as plsc

_FEATURES = 64
_ROWS_PER_STEP = 4


def kernel(inputs, embedding):
    batch, seq = inputs.shape
    seq_pad = 56

    pad_cols = 128 - seq
    filler = (
        jax.lax.broadcasted_iota(jnp.int32, (batch, pad_cols), 0) * pad_cols
        + jax.lax.broadcasted_iota(jnp.int32, (batch, pad_cols), 1)
    )
    idx = jnp.concatenate([inputs.astype(jnp.int32), filler], axis=1)
    idx = idx.reshape(batch * 128)

    table = jnp.pad(embedding, ((0, 0), (0, 128 - _FEATURES)))

    mesh = plsc.VectorSubcoreMesh(
        core_axis_name="core", subcore_axis_name="subcore"
    )

    @pl.kernel(
        out_type=jax.ShapeDtypeStruct((batch * seq_pad, 128), embedding.dtype),
        mesh=mesh,
        scratch_types=[pltpu.SemaphoreType.DMA],
        compiler_params=pltpu.CompilerParams(use_tc_tiling_on_sc=False),
    )
    def _gather(x_hbm, i_hbm, o_hbm, sem):
        def body(i_vmem, o_vmem):
            copies = [
                pltpu.async_copy(
                    x_hbm.at[i_vmem.at[pl.ds(r * 128, seq_pad)]],
                    o_vmem.at[pl.ds(r * seq_pad, seq_pad)],
                    sem,
                )
                for r in range(_ROWS_PER_STEP)
            ]
            for c in copies:
                c.wait()

        pltpu.emit_pipeline(
            body,
            grid=(batch // _ROWS_PER_STEP,),
            in_specs=[
                pl.BlockSpec(
                    (_ROWS_PER_STEP * 128,), index_map=lambda i: (i,)
                )
            ],
            out_specs=[
                pl.BlockSpec(
                    (_ROWS_PER_STEP * seq_pad, 128),
                    index_map=lambda i: (i, 0),
                )
            ],
            core_axis_name=("core", "subcore"),
            dimension_semantics=(pltpu.PARALLEL,),
        )(i_hbm, o_hbm)

    out = _gather(table, idx)
    return out.reshape(batch, seq_pad, 128)[:, :seq, :_FEATURES]


# speedup vs baseline: 1.1734x; 1.0023x over previous
import jax
import jax.numpy as jnp
from jax.experimental import pallas as pl
from jax.experimental.pallas import tpu as pltpu
from jax.experimental.pallas import tpu_sc as plsc

_FEATURES = 64
_ROWS_PER_STEP = 8


def kernel(inputs, embedding):
    batch, seq = inputs.shape
    seq_pad = 56

    pad_cols = 128 - seq
    filler = (
        jax.lax.broadcasted_iota(jnp.int32, (batch, pad_cols), 0) * pad_cols
        + jax.lax.broadcasted_iota(jnp.int32, (batch, pad_cols), 1)
    )
    idx = jnp.concatenate([inputs.astype(jnp.int32), filler], axis=1)
    idx = idx.reshape(batch * 128)

    table = jnp.pad(embedding, ((0, 0), (0, 128 - _FEATURES)))

    mesh = plsc.VectorSubcoreMesh(
        core_axis_name="core", subcore_axis_name="subcore"
    )

    @pl.kernel(
        out_type=jax.ShapeDtypeStruct((batch * seq_pad, 128), embedding.dtype),
        mesh=mesh,
        scratch_types=[pltpu.SemaphoreType.DMA],
        compiler_params=pltpu.CompilerParams(use_tc_tiling_on_sc=False),
    )
    def _gather(x_hbm, i_hbm, o_hbm, sem):
        def body(i_vmem, o_vmem):
            copies = [
                pltpu.async_copy(
                    x_hbm.at[i_vmem.at[pl.ds(r * 128, seq_pad)]],
                    o_vmem.at[pl.ds(r * seq_pad, seq_pad)],
                    sem,
                )
                for r in range(_ROWS_PER_STEP)
            ]
            for c in copies:
                c.wait()

        pltpu.emit_pipeline(
            body,
            grid=(batch // _ROWS_PER_STEP,),
            in_specs=[
                pl.BlockSpec(
                    (_ROWS_PER_STEP * 128,), index_map=lambda i: (i,)
                )
            ],
            out_specs=[
                pl.BlockSpec(
                    (_ROWS_PER_STEP * seq_pad, 128),
                    index_map=lambda i: (i, 0),
                )
            ],
            core_axis_name=("core", "subcore"),
            dimension_semantics=(pltpu.PARALLEL,),
        )(i_hbm, o_hbm)

    out = _gather(table, idx)
    return out.reshape(batch, seq_pad, 128)[:, :seq, :_FEATURES]
